# baseline (speedup 1.0000x reference)
import functools

import jax
import jax.numpy as jnp
from jax import lax
from jax.experimental import pallas as pl
from jax.experimental.pallas import tpu as pltpu
from jax.experimental.pallas import tpu_sc as plsc

D = 128
N = 1_000_000
B = 16384
NUM_CORES = 2
NUM_SUBCORES = 16
NW = NUM_CORES * NUM_SUBCORES
ROWS_PER_TILE = D // NW
CHUNK = 128
NCHUNKS = B // CHUNK
LANES = 16


def _sc_gather(mem_flat, idx):
    mesh = plsc.VectorSubcoreMesh(core_axis_name="c", subcore_axis_name="s")

    @functools.partial(
        pl.kernel,
        mesh=mesh,
        out_type=jax.ShapeDtypeStruct((D, B), jnp.float32),
        scratch_types=[
            pltpu.VMEM((B,), jnp.int32),
            pltpu.VMEM((B,), jnp.int32),
            pltpu.VMEM((B,), jnp.float32),
            pltpu.SemaphoreType.DMA,
        ],
    )
    def k(mem_hbm, idx_hbm, out_hbm, idx_v, fidx_v, row_v, sem):
        wid = lax.axis_index("s") * NUM_CORES + lax.axis_index("c")
        pltpu.sync_copy(idx_hbm, idx_v)
        for r in range(ROWS_PER_TILE):
            d = wid * ROWS_PER_TILE + r
            base = d * N

            def add_body(i, carry):
                fidx_v[pl.ds(i * LANES, LANES)] = (
                    idx_v[pl.ds(i * LANES, LANES)] + base
                )
                return carry

            lax.fori_loop(0, B // LANES, add_body, 0)

            def fire(c, carry):
                pltpu.async_copy(
                    mem_hbm.at[fidx_v.at[pl.ds(c * CHUNK, CHUNK)]],
                    row_v.at[pl.ds(c * CHUNK, CHUNK)],
                    sem,
                )
                return carry

            lax.fori_loop(0, NCHUNKS, fire, 0)

            pltpu.make_async_copy(mem_hbm.at[pl.ds(0, B)], row_v, sem).wait()
            pltpu.sync_copy(row_v, out_hbm.at[d])

    return k(mem_flat, idx)


def kernel(membank, n_index):
    mem_flat = membank.reshape(-1)
    idx = n_index.astype(jnp.int32)
    return _sc_gather(mem_flat, idx)
